# baseline (speedup 1.0000x reference)
import functools

import jax
import jax.numpy as jnp
from jax import lax
from jax.experimental import pallas as pl
from jax.experimental.pallas import tpu as pltpu
from jax.experimental.pallas import tpu_sc as plsc

_info = plsc.get_sparse_core_info()
_NC, _NS = _info.num_cores, _info.num_subcores
_NW = _NC * _NS


def _make_gather(B: int, V: int, D: int, ch: int):
    b_per_w = B // _NW
    n_ch = b_per_w // ch
    mesh = plsc.VectorSubcoreMesh(core_axis_name="c", subcore_axis_name="s")

    @functools.partial(
        pl.kernel,
        mesh=mesh,
        out_type=jax.ShapeDtypeStruct((B, D), jnp.float32),
        scratch_types=[
            pltpu.VMEM((ch,), jnp.int32),
            pltpu.VMEM((ch, D), jnp.float32),
            pltpu.SemaphoreType.DMA,
        ],
        compiler_params=pltpu.CompilerParams(use_tc_tiling_on_sc=False),
    )
    def k(idx_hbm, w_hbm, out_hbm, idx_v, rows_v, sem):
        wid = lax.axis_index("s") * _NC + lax.axis_index("c")
        base_w = wid * b_per_w

        def body(i, carry):
            base = base_w + i * ch
            pltpu.sync_copy(idx_hbm.at[pl.ds(base, ch)], idx_v)
            pltpu.async_copy(w_hbm.at[idx_v], rows_v, sem).wait()
            pltpu.sync_copy(rows_v, out_hbm.at[pl.ds(base, ch)])
            return carry

        lax.fori_loop(0, n_ch, body, 0)

    return k


def kernel(input_, weight):
    bsz, seq = input_.shape
    V, D = weight.shape
    idx = input_.reshape(-1).astype(jnp.int32)
    B = idx.shape[0]
    out = _make_gather(B, V, D, ch=128)(idx, weight)
    return out.reshape(bsz, seq, D)


# speedup vs baseline: 1.1683x; 1.1683x over previous
import functools

import jax
import jax.numpy as jnp
from jax import lax
from jax.experimental import pallas as pl
from jax.experimental.pallas import tpu as pltpu
from jax.experimental.pallas import tpu_sc as plsc

_info = plsc.get_sparse_core_info()
_NC, _NS = _info.num_cores, _info.num_subcores
_NW = _NC * _NS


def _make_gather(B: int, V: int, D: int, ch: int):
    b_per_w = B // _NW
    n_ch = b_per_w // ch
    mesh = plsc.VectorSubcoreMesh(core_axis_name="c", subcore_axis_name="s")

    @functools.partial(
        pl.kernel,
        mesh=mesh,
        out_type=jax.ShapeDtypeStruct((B, D), jnp.float32),
        scratch_types=[
            pltpu.VMEM((n_ch, ch), jnp.int32),
            pltpu.VMEM((ch, D), jnp.float32),
            pltpu.SemaphoreType.DMA,
        ],
        compiler_params=pltpu.CompilerParams(use_tc_tiling_on_sc=False),
    )
    def k(idx_hbm, w_hbm, out_hbm, idx_v, rows_v, sem):
        wid = lax.axis_index("s") * _NC + lax.axis_index("c")
        base_w = wid * b_per_w
        pltpu.sync_copy(idx_hbm.at[wid], idx_v)

        def body(i, carry):
            pltpu.async_copy(w_hbm.at[idx_v.at[i]], rows_v, sem).wait()
            pltpu.sync_copy(rows_v, out_hbm.at[pl.ds(base_w + i * ch, ch)])
            return carry

        lax.fori_loop(0, n_ch, body, 0)

    return k


def kernel(input_, weight):
    bsz, seq = input_.shape
    V, D = weight.shape
    idx = input_.reshape(-1).astype(jnp.int32)
    B = idx.shape[0]
    ch = 512
    idx3 = idx.reshape(_NW, (B // _NW) // ch, ch)
    out = _make_gather(B, V, D, ch=ch)(idx3, weight)
    return out.reshape(bsz, seq, D)


# speedup vs baseline: 1.1952x; 1.0230x over previous
import functools

import jax
import jax.numpy as jnp
from jax import lax
from jax.experimental import pallas as pl
from jax.experimental.pallas import tpu as pltpu
from jax.experimental.pallas import tpu_sc as plsc

_info = plsc.get_sparse_core_info()
_NC, _NS = _info.num_cores, _info.num_subcores
_NW = _NC * _NS


def _make_gather(B: int, V: int, D: int, ch: int):
    b_per_w = B // _NW
    n_ch = b_per_w // ch
    assert n_ch % 2 == 0 and n_ch >= 4
    mesh = plsc.VectorSubcoreMesh(core_axis_name="c", subcore_axis_name="s")

    @functools.partial(
        pl.kernel,
        mesh=mesh,
        out_type=jax.ShapeDtypeStruct((B, D), jnp.float32),
        scratch_types=[
            pltpu.VMEM((n_ch, ch), jnp.int32),
            pltpu.VMEM((2, ch, D), jnp.float32),
            pltpu.SemaphoreType.DMA,
            pltpu.SemaphoreType.DMA,
            pltpu.SemaphoreType.DMA,
            pltpu.SemaphoreType.DMA,
        ],
        compiler_params=pltpu.CompilerParams(use_tc_tiling_on_sc=False),
    )
    def k(idx_hbm, w_hbm, out_hbm, idx_v, rows_v, sg0, sg1, so0, so1):
        wid = lax.axis_index("s") * _NC + lax.axis_index("c")
        base_w = wid * b_per_w
        sg = (sg0, sg1)
        so = (so0, so1)

        def fire_g(c, b):
            pltpu.async_copy(w_hbm.at[idx_v.at[c]], rows_v.at[b], sg[b])

        def wait_g(c, b):
            pltpu.make_async_copy(
                w_hbm.at[idx_v.at[c]], rows_v.at[b], sg[b]
            ).wait()

        def fire_o(c, b):
            pltpu.async_copy(
                rows_v.at[b], out_hbm.at[pl.ds(base_w + c * ch, ch)], so[b]
            )

        def wait_o(c, b):
            pltpu.make_async_copy(
                rows_v.at[b], out_hbm.at[pl.ds(base_w + c * ch, ch)], so[b]
            ).wait()

        pltpu.sync_copy(idx_hbm.at[wid], idx_v)

        fire_g(0, 0)
        fire_g(1, 1)
        wait_g(0, 0)
        fire_o(0, 0)

        def body(g, carry):
            for (c, b) in ((2 * g + 1, 1), (2 * g + 2, 0)):
                wait_o(c - 1, 1 - b)
                fire_g(c + 1, 1 - b)
                wait_g(c, b)
                fire_o(c, b)
            return carry

        lax.fori_loop(0, (n_ch - 2) // 2, body, 0)

        c = n_ch - 1
        wait_o(c - 1, 0)
        wait_g(c, 1)
        fire_o(c, 1)
        wait_o(c, 1)

    return k


def kernel(input_, weight):
    bsz, seq = input_.shape
    V, D = weight.shape
    idx = input_.reshape(-1).astype(jnp.int32)
    B = idx.shape[0]
    ch = 512
    idx3 = idx.reshape(_NW, (B // _NW) // ch, ch)
    out = _make_gather(B, V, D, ch=ch)(idx3, weight)
    return out.reshape(bsz, seq, D)
